# baseline (speedup 1.0000x reference)
import jax
import jax.numpy as jnp
from jax import lax
from jax.experimental import pallas as pl
from jax.experimental.pallas import tpu as pltpu
from jax.experimental.pallas import tpu_sc as plsc

_P = 4
_ALPHA = 1.0
_ZCLIP = 13.8155096

_LANE = 128
_B = 8

_NC, _NS, _SCL = 2, 16, 16
_NW = _NC * _NS
_N_ANCH = 12288
_SPAN = 2 * _N_ANCH // _NW


def _sc_body(ac_hbm, idx_out, cnt_out, ac_v, idx_v, cnt_v):
    wid = lax.axis_index("s") * _NC + lax.axis_index("c")
    base = wid * _SPAN
    pltpu.sync_copy(ac_hbm.at[pl.ds(base, _SPAN)], ac_v)
    local0 = base - (base // _N_ANCH) * _N_ANCH
    woff = jnp.zeros((_SCL,), jnp.int32)
    lane = lax.iota(jnp.int32, _SCL)
    for c in range(_SPAN // _SCL):
        vals = ac_v[pl.ds(c * _SCL, _SCL)]
        m = vals != 0.0
        pc = plsc.cumsum(m.astype(jnp.int32))
        dest = woff + pc - 1
        flat = lane + (local0 + c * _SCL)
        plsc.store_scatter(idx_v, [dest], flat, mask=m)
        woff = woff + plsc.all_reduce_population_count(m)
    cnt_v[...] = woff
    pltpu.sync_copy(idx_v, idx_out.at[wid])
    pltpu.sync_copy(cnt_v, cnt_out.at[wid])


def _sc_compact(ac_flat):
    mesh = plsc.VectorSubcoreMesh(core_axis_name="c", subcore_axis_name="s")
    f = pl.kernel(
        _sc_body,
        mesh=mesh,
        compiler_params=pltpu.CompilerParams(needs_layout_passes=False),
        out_type=(
            jax.ShapeDtypeStruct((_NW, _SPAN), jnp.int32),
            jax.ShapeDtypeStruct((_NW, _SCL), jnp.int32),
        ),
        scratch_types=[
            pltpu.VMEM((_SPAN,), jnp.float32),
            pltpu.VMEM((_SPAN,), jnp.int32),
            pltpu.VMEM((_SCL,), jnp.int32),
        ],
    )
    return f(ac_flat)


def _tc_kernel(proto_ref, cls_ref, box_ref, coef_ref, ctr_ref, abox_ref,
               gtb_ref, gtm_ref, ac_ref, g4_ref, idx_ref, cnt_ref, out_ref):
    n_img = proto_ref.shape[0]
    n_rows = ac_ref.shape[1]
    n_anch = n_rows * _LANE
    a_num = g4_ref.shape[1]
    rows_per_a = g4_ref.shape[2]
    n_px = proto_ref.shape[2]
    inv_px = 1.0 / float(n_px)
    inv_ln10 = 0.43429448190325176
    slots = idx_ref.shape[1]
    slot_rows = idx_ref.shape[2]

    r0 = lax.broadcasted_iota(jnp.int32, (_LANE, _LANE), 0)
    c0 = lax.broadcasted_iota(jnp.int32, (_LANE, _LANE), 1)
    tri_incl = (r0 <= c0).astype(jnp.float32)
    rr0 = lax.broadcasted_iota(jnp.int32, (n_rows, n_rows), 0)
    cc0 = lax.broadcasted_iota(jnp.int32, (n_rows, n_rows), 1)
    tri_strict = (cc0 < rr0).astype(jnp.float32)

    lane_iota = lax.broadcasted_iota(jnp.int32, (1, _LANE), 1)
    lane3 = lax.broadcasted_iota(jnp.int32, (_P, 1, _LANE), 2)
    iota_b = lax.broadcasted_iota(jnp.int32, (1, _B), 1)

    total = jnp.float32(0.0)
    for i in range(n_img):
        ac = ac_ref[i]
        posf = (ac != 0).astype(jnp.float32)
        negf = 1.0 - posf
        npos = jnp.sum(posf)
        has_pos = npos > 0.0
        npos_f = jnp.where(has_pos, npos, 1.0)
        total_neg = float(n_anch) - npos
        nneg = jnp.minimum(3.0 * npos, total_neg)
        nneg_f = jnp.where(nneg > 0.0, nneg, 1.0)

        rc_neg = lax.dot(negf, tri_incl,
                         precision=lax.Precision.HIGHEST,
                         preferred_element_type=jnp.float32)
        offs_neg = lax.dot(tri_strict, rc_neg[:, _LANE - 1:_LANE],
                           precision=lax.Precision.HIGHEST,
                           preferred_element_type=jnp.float32)
        negcum = rc_neg + offs_neg

        p = cls_ref[i]
        p_c = jnp.clip(p, 1e-6, 1.0 - 1e-6)
        neg_sel = negf * (negcum <= nneg).astype(jnp.float32)
        l_cls_pos = jnp.sum(posf * (-jnp.log(p_c))) / npos_f
        l_cls_neg = jnp.sum(neg_sel * (-jnp.log(1.0 - p_c))) / nneg_f

        ach = ctr_ref[0]
        acw = ctr_ref[1]
        l_loc = jnp.float32(0.0)
        for a in range(a_num):
            g_a = g4_ref[i, a]
            pos_a = posf[a * rows_per_a:(a + 1) * rows_per_a, :]
            a_h = abox_ref[a, 0]
            a_w = abox_ref[a, 1]
            gts = []
            for x in range(4):
                acc = jnp.zeros((rows_per_a, _LANE), jnp.float32)
                for j in range(gtb_ref.shape[1]):
                    acc = jnp.where(g_a == j, gtb_ref[i, j, x], acc)
                gts.append(acc)
            t0 = (gts[0] - ach) / a_h
            t1 = (gts[1] - acw) / a_w
            t2 = jnp.log(gts[2] / a_h) * inv_ln10
            t3 = jnp.log(gts[3] / a_w) * inv_ln10
            for x, t in enumerate((t0, t1, t2, t3)):
                pr = box_ref[i, a * 4 + x]
                d = jnp.abs(pr - t)
                sl = jnp.where(d < 1.0, 0.5 * d * d, d - 0.5)
                l_loc = l_loc + jnp.sum(pos_a * sl)

        proto_i = proto_ref[i]
        l_msk = jnp.float32(0.0)
        for s in range(slots):
            cnt = cnt_ref[i, s, 0]
            nch = (cnt + (_B - 1)) // _B

            def chunk_body(c, acc, s=s, cnt=cnt):
                krow = idx_ref[i, s, pl.ds(c, 1), :]
                base = c * _B
                crows = []
                yrows = []
                vrows = []
                for jj in range(_B):
                    k = jnp.sum(jnp.where(iota_b == jj, krow, 0))
                    k = jnp.clip(k, 0, n_anch - 1)
                    a = k // (rows_per_a * _LANE)
                    rem = k - a * (rows_per_a * _LANE)
                    rw = rem // _LANE
                    cl = rem - rw * _LANE
                    oh = lane_iota == cl
                    grow = g4_ref[i, a, pl.ds(rw, 1), :]
                    g = jnp.sum(jnp.where(oh, grow, 0))
                    cblk = coef_ref[i, pl.ds(a * _P, _P), pl.ds(rw, 1), :]
                    cvec = jnp.sum(jnp.where(lane3 == cl, cblk, 0.0),
                                   axis=2, keepdims=True)
                    crows.append(cvec.reshape(_P, 1))
                    yrows.append(gtm_ref[i, pl.ds(g, 1), :])
                    valid = (base + jj < cnt).astype(jnp.float32)
                    vrows.append(valid.reshape(1, 1))
                cmat_t = jnp.concatenate(crows, axis=1)
                ymat = jnp.concatenate(yrows, axis=0)
                vmat = jnp.concatenate(vrows, axis=0)
                z = lax.dot_general(
                    cmat_t, proto_i, (((0,), (0,)), ((), ())),
                    precision=lax.Precision.HIGHEST,
                    preferred_element_type=jnp.float32)
                zc = jnp.clip(z, -_ZCLIP, _ZCLIP)
                sp = jnp.maximum(zc, 0.0) + jnp.log1p(jnp.exp(-jnp.abs(zc)))
                t = sp - ymat * zc
                rs = jnp.sum(t, axis=1, keepdims=True)
                return acc + jnp.sum(vmat * rs)

            l_msk = lax.fori_loop(0, nch, chunk_body, l_msk)
        l_msk = l_msk * inv_px

        total = total + jnp.where(
            has_pos,
            (l_cls_pos + l_cls_neg) / npos_f
            + _ALPHA * l_loc / npos_f
            + l_msk / npos_f,
            0.0)

    out_ref[:, :] = jnp.broadcast_to(total, (1, 1))


def kernel(proto_types, map_class, map_box, map_coef, anchor_center,
           anchor_box, gt_boxes, gt_masks, anchor_class, gt_idx):
    n, a_num, h, w = anchor_class.shape
    n_rows = a_num * h * w // _LANE
    rows_per_a = h * w // _LANE
    n_px = proto_types.shape[2] * proto_types.shape[3]

    idx, cnts = _sc_compact(anchor_class.reshape(-1))
    idx4 = idx.reshape(n, _NW // n, _SPAN // _B, _B)
    cnt3 = cnts.reshape(n, _NW // n, _SCL)

    proto2 = proto_types.reshape(n, _P, n_px)
    cls2 = map_class.reshape(n, n_rows, _LANE)
    box4 = map_box.reshape(n, a_num * 4, rows_per_a, _LANE)
    coef4 = map_coef.reshape(n, a_num * _P, rows_per_a, _LANE)
    ctr = anchor_center.reshape(2, rows_per_a, _LANE)
    ac2 = anchor_class.reshape(n, n_rows, _LANE)
    g4 = gt_idx.reshape(n, a_num, rows_per_a, _LANE)
    gtm2 = gt_masks.reshape(n, gt_masks.shape[1], n_px)

    out = pl.pallas_call(
        _tc_kernel,
        out_shape=jax.ShapeDtypeStruct((1, 1), jnp.float32),
    )(proto2, cls2, box4, coef4, ctr, anchor_box, gt_boxes,
      gtm2, ac2, g4, idx4, cnt3)
    return out.reshape(())


# speedup vs baseline: 1.0458x; 1.0458x over previous
import jax
import jax.numpy as jnp
from jax import lax
from jax.experimental import pallas as pl
from jax.experimental.pallas import tpu as pltpu
from jax.experimental.pallas import tpu_sc as plsc

_P = 4
_ALPHA = 1.0
_ZCLIP = 13.8155096

_LANE = 128
_B = 8

_NC, _NS, _SCL = 2, 16, 16
_NW = _NC * _NS
_N_ANCH = 12288
_SPAN = 2 * _N_ANCH // _NW


def _sc_body(ac_hbm, idx_out, cnt_out, ac_v, idx_v, cnt_v):
    wid = lax.axis_index("s") * _NC + lax.axis_index("c")
    base = wid * _SPAN
    pltpu.sync_copy(ac_hbm.at[pl.ds(base, _SPAN)], ac_v)
    local0 = base - (base // _N_ANCH) * _N_ANCH
    woff = jnp.zeros((_SCL,), jnp.int32)
    lane = lax.iota(jnp.int32, _SCL)
    for c in range(_SPAN // _SCL):
        vals = ac_v[pl.ds(c * _SCL, _SCL)]
        m = vals != 0.0
        pc = plsc.cumsum(m.astype(jnp.int32))
        dest = woff + pc - 1
        flat = lane + (local0 + c * _SCL)
        plsc.store_scatter(idx_v, [dest], flat, mask=m)
        woff = woff + plsc.all_reduce_population_count(m)
    cnt_v[...] = woff
    pltpu.sync_copy(idx_v, idx_out.at[wid])
    pltpu.sync_copy(cnt_v, cnt_out.at[wid])


def _sc_compact(ac_flat):
    mesh = plsc.VectorSubcoreMesh(core_axis_name="c", subcore_axis_name="s")
    f = pl.kernel(
        _sc_body,
        mesh=mesh,
        compiler_params=pltpu.CompilerParams(needs_layout_passes=False),
        out_type=(
            jax.ShapeDtypeStruct((_NW, _SPAN), jnp.int32),
            jax.ShapeDtypeStruct((_NW, _SCL), jnp.int32),
        ),
        scratch_types=[
            pltpu.VMEM((_SPAN,), jnp.float32),
            pltpu.VMEM((_SPAN,), jnp.int32),
            pltpu.VMEM((_SCL,), jnp.int32),
        ],
    )
    return f(ac_flat)


def _tc_kernel(proto_ref, cls_ref, box_ref, coef_ref, ctr_ref, abox_ref,
               gtb_ref, gtm_ref, ac_ref, g4_ref, idx_ref, cnt_ref, out_ref):
    n_img = proto_ref.shape[0]
    n_rows = ac_ref.shape[1]
    n_anch = n_rows * _LANE
    a_num = g4_ref.shape[1]
    rows_per_a = g4_ref.shape[2]
    n_px = proto_ref.shape[2] * proto_ref.shape[3]
    inv_px = 1.0 / float(n_px)
    inv_ln10 = 0.43429448190325176
    slots = idx_ref.shape[1]
    slot_rows = idx_ref.shape[2]

    r0 = lax.broadcasted_iota(jnp.int32, (_LANE, _LANE), 0)
    c0 = lax.broadcasted_iota(jnp.int32, (_LANE, _LANE), 1)
    tri_incl = (r0 <= c0).astype(jnp.float32)
    rr0 = lax.broadcasted_iota(jnp.int32, (n_rows, n_rows), 0)
    cc0 = lax.broadcasted_iota(jnp.int32, (n_rows, n_rows), 1)
    tri_strict = (cc0 < rr0).astype(jnp.float32)

    lane_iota = lax.broadcasted_iota(jnp.int32, (1, _LANE), 1)
    lane3 = lax.broadcasted_iota(jnp.int32, (_P, 1, _LANE), 2)
    iota_b = lax.broadcasted_iota(jnp.int32, (1, _B), 1)

    total = jnp.float32(0.0)
    for i in range(n_img):
        ac = ac_ref[i]
        posf = (ac != 0).astype(jnp.float32)
        negf = 1.0 - posf
        npos = jnp.sum(posf)
        has_pos = npos > 0.0
        npos_f = jnp.where(has_pos, npos, 1.0)
        total_neg = float(n_anch) - npos
        nneg = jnp.minimum(3.0 * npos, total_neg)
        nneg_f = jnp.where(nneg > 0.0, nneg, 1.0)

        rc_neg = lax.dot(negf, tri_incl,
                         precision=lax.Precision.HIGHEST,
                         preferred_element_type=jnp.float32)
        offs_neg = lax.dot(tri_strict, rc_neg[:, _LANE - 1:_LANE],
                           precision=lax.Precision.HIGHEST,
                           preferred_element_type=jnp.float32)
        negcum = rc_neg + offs_neg

        p = cls_ref[i]
        p_c = jnp.clip(p, 1e-6, 1.0 - 1e-6)
        neg_sel = negf * (negcum <= nneg).astype(jnp.float32)
        l_cls_pos = jnp.sum(posf * (-jnp.log(p_c))) / npos_f
        l_cls_neg = jnp.sum(neg_sel * (-jnp.log(1.0 - p_c))) / nneg_f

        ach = ctr_ref[0]
        acw = ctr_ref[1]
        l_loc = jnp.float32(0.0)
        for a in range(a_num):
            g_a = g4_ref[i, a]
            pos_a = posf[a * rows_per_a:(a + 1) * rows_per_a, :]
            a_h = abox_ref[a, 0]
            a_w = abox_ref[a, 1]
            gts = []
            for x in range(4):
                acc = jnp.zeros((rows_per_a, _LANE), jnp.float32)
                for j in range(gtb_ref.shape[1]):
                    acc = jnp.where(g_a == j, gtb_ref[i, j, x], acc)
                gts.append(acc)
            t0 = (gts[0] - ach) / a_h
            t1 = (gts[1] - acw) / a_w
            t2 = jnp.log(gts[2] / a_h) * inv_ln10
            t3 = jnp.log(gts[3] / a_w) * inv_ln10
            for x, t in enumerate((t0, t1, t2, t3)):
                pr = box_ref[i, a * 4 + x]
                d = jnp.abs(pr - t)
                sl = jnp.where(d < 1.0, 0.5 * d * d, d - 0.5)
                l_loc = l_loc + jnp.sum(pos_a * sl)

        protos = [proto_ref[i, pp] for pp in range(_P)]
        l_msk = jnp.float32(0.0)
        for s in range(slots):
            cnt = cnt_ref[i, s, 0]

            def pos_body(j, acc, s=s):
                krow = idx_ref[i, s, pl.ds(j // _B, 1), :]
                jl = j - (j // _B) * _B
                k = jnp.sum(jnp.where(iota_b == jl, krow, 0))
                k = jnp.clip(k, 0, n_anch - 1)
                a = k // (rows_per_a * _LANE)
                rem = k - a * (rows_per_a * _LANE)
                rw = rem // _LANE
                cl = rem - rw * _LANE
                oh = lane_iota == cl
                grow = g4_ref[i, a, pl.ds(rw, 1), :]
                g = jnp.sum(jnp.where(oh, grow, 0))
                cblk = coef_ref[i, pl.ds(a * _P, _P), pl.ds(rw, 1), :]
                cvec = jnp.sum(jnp.where(lane3 == cl, cblk, 0.0),
                               axis=2, keepdims=True)
                cvec = cvec.reshape(_P, 1)
                z = (cvec[0, 0] * protos[0] + cvec[1, 0] * protos[1]
                     + cvec[2, 0] * protos[2] + cvec[3, 0] * protos[3])
                zc = jnp.clip(z, -_ZCLIP, _ZCLIP)
                sp = jnp.maximum(zc, 0.0) + jnp.log1p(jnp.exp(-jnp.abs(zc)))
                y = gtm_ref[i, g]
                return acc + (jnp.sum(sp) - jnp.sum(y * zc))

            l_msk = lax.fori_loop(0, cnt, pos_body, l_msk)
        l_msk = l_msk * inv_px

        total = total + jnp.where(
            has_pos,
            (l_cls_pos + l_cls_neg) / npos_f
            + _ALPHA * l_loc / npos_f
            + l_msk / npos_f,
            0.0)

    out_ref[:, :] = jnp.broadcast_to(total, (1, 1))


def kernel(proto_types, map_class, map_box, map_coef, anchor_center,
           anchor_box, gt_boxes, gt_masks, anchor_class, gt_idx):
    n, a_num, h, w = anchor_class.shape
    n_rows = a_num * h * w // _LANE
    rows_per_a = h * w // _LANE
    n_px = proto_types.shape[2] * proto_types.shape[3]

    idx, cnts = _sc_compact(anchor_class.reshape(-1))
    idx4 = idx.reshape(n, _NW // n, _SPAN // _B, _B)
    cnt3 = cnts.reshape(n, _NW // n, _SCL)

    proto2 = proto_types
    cls2 = map_class.reshape(n, n_rows, _LANE)
    box4 = map_box.reshape(n, a_num * 4, rows_per_a, _LANE)
    coef4 = map_coef.reshape(n, a_num * _P, rows_per_a, _LANE)
    ctr = anchor_center.reshape(2, rows_per_a, _LANE)
    ac2 = anchor_class.reshape(n, n_rows, _LANE)
    g4 = gt_idx.reshape(n, a_num, rows_per_a, _LANE)
    gtm2 = gt_masks

    out = pl.pallas_call(
        _tc_kernel,
        out_shape=jax.ShapeDtypeStruct((1, 1), jnp.float32),
    )(proto2, cls2, box4, coef4, ctr, anchor_box, gt_boxes,
      gtm2, ac2, g4, idx4, cnt3)
    return out.reshape(())
